# baseline (speedup 1.0000x reference)
import functools

import jax
import jax.numpy as jnp
from jax import lax
from jax.experimental import pallas as pl
from jax.experimental.pallas import tpu as pltpu
from jax.experimental.pallas import tpu_sc as plsc

N_NODES = 10000
D = 128
N_EDGES = 320000

NC = 2
NS = 16
NW = NC * NS
CHUNK = 128
NBUF = 2
NCHUNK = 80
WIN = 16
NWIN = NCHUNK // WIN
E_PAD = NW * NCHUNK * CHUNK
DUMMY = N_NODES
N_PAD = 10240
ROWS_PER_TILE = N_PAD // NS

_mesh = plsc.VectorSubcoreMesh(core_axis_name="c", subcore_axis_name="s")


@functools.partial(
    pl.kernel,
    out_type=jax.ShapeDtypeStruct((NW, N_PAD), jnp.float32),
    mesh=_mesh,
    scratch_types=[
        pltpu.VMEM((NCHUNK, CHUNK), jnp.int32),
        pltpu.VMEM((N_PAD,), jnp.float32),
    ],
    compiler_params=pltpu.CompilerParams(needs_layout_passes=False),
)
def _deg_kernel(dst_hbm, out_hbm, dst_v, deg_v):
    cid = lax.axis_index("c")
    sid = lax.axis_index("s")
    wid = cid * NS + sid
    pltpu.sync_copy(dst_hbm.at[wid], dst_v)

    zeros16 = jnp.zeros((16,), jnp.float32)

    def _zero(r, carry):
        deg_v[pl.ds(r * 16, 16)] = zeros16
        return carry

    lax.fori_loop(0, N_PAD // 16, _zero, 0)

    ones16 = jnp.ones((16,), jnp.float32)

    def _edges(j, carry):
        def _sub(k, c2):
            idx = dst_v[j, pl.ds(k * 16, 16)]
            plsc.addupdate_scatter(deg_v, [idx], ones16)
            return c2
        return lax.fori_loop(0, CHUNK // 16, _sub, carry)

    lax.fori_loop(0, NCHUNK, _edges, 0)
    pltpu.sync_copy(deg_v, out_hbm.at[wid])


@functools.partial(
    pl.kernel,
    out_type=jax.ShapeDtypeStruct((NC, N_PAD, D), jnp.float32),
    mesh=_mesh,
    scratch_types=[
        pltpu.VMEM((WIN, CHUNK), jnp.int32),
        pltpu.VMEM((WIN, CHUNK), jnp.int32),
        pltpu.VMEM((WIN, CHUNK), jnp.int32),
        pltpu.VMEM((WIN, CHUNK), jnp.int32),
        pltpu.VMEM((CHUNK, D), jnp.float32),
        pltpu.VMEM((CHUNK, D), jnp.float32),
        pltpu.VMEM_SHARED((N_PAD, D), jnp.float32),
        pltpu.SemaphoreType.DMA,
        pltpu.SemaphoreType.DMA,
        pltpu.SemaphoreType.DMA,
        pltpu.SemaphoreType.DMA,
    ],
    compiler_params=pltpu.CompilerParams(needs_layout_passes=False),
)
def _scatter_kernel(u_hbm, src_hbm, dst_hbm, out_hbm,
                    sw0, sw1, dw0, dw1, r0b, r1b, acc, s0, s1, ws0, ws1):
    swin = (sw0, sw1)
    dwin = (dw0, dw1)
    rows = (r0b, r1b)
    sems = (s0, s1)
    wsems = (ws0, ws1)
    cid = lax.axis_index("c")
    sid = lax.axis_index("s")
    wid = cid * NS + sid
    r0 = sid * ROWS_PER_TILE
    pltpu.async_copy(src_hbm.at[wid, pl.ds(0, WIN)], swin[0], wsems[0])
    pltpu.async_copy(dst_hbm.at[wid, pl.ds(0, WIN)], dwin[0], wsems[0])
    pltpu.sync_copy(u_hbm.at[pl.ds(r0, ROWS_PER_TILE)],
                    acc.at[pl.ds(r0, ROWS_PER_TILE)])
    plsc.subcore_barrier()
    pltpu.make_async_copy(src_hbm.at[wid, pl.ds(0, WIN)], swin[0],
                          wsems[0]).wait()
    pltpu.make_async_copy(dst_hbm.at[wid, pl.ds(0, WIN)], dwin[0],
                          wsems[0]).wait()

    for w in range(NWIN):
        slot = w % 2
        nslot = 1 - slot
        if w + 1 < NWIN:
            pltpu.async_copy(src_hbm.at[wid, pl.ds((w + 1) * WIN, WIN)],
                             swin[nslot], wsems[nslot])
            pltpu.async_copy(dst_hbm.at[wid, pl.ds((w + 1) * WIN, WIN)],
                             dwin[nslot], wsems[nslot])
        sv = swin[slot]
        dv = dwin[slot]
        for b in range(NBUF):
            pltpu.async_copy(u_hbm.at[sv.at[b]], rows[b], sems[b])

        def _group(g, carry, sv=sv, dv=dv):
            for b in range(NBUF):
                r = g * NBUF + b
                pltpu.make_async_copy(u_hbm.at[sv.at[r]], rows[b],
                                      sems[b]).wait()
                pltpu.sync_copy(rows[b], acc.at[dv.at[r]], add=True)
                pltpu.async_copy(u_hbm.at[sv.at[r + NBUF]], rows[b], sems[b])
            return carry

        lax.fori_loop(0, WIN // NBUF - 1, _group, 0)
        for b in range(NBUF):
            r = WIN - NBUF + b
            pltpu.make_async_copy(u_hbm.at[sv.at[r]], rows[b],
                                  sems[b]).wait()
            pltpu.sync_copy(rows[b], acc.at[dv.at[r]], add=True)
        if w + 1 < NWIN:
            pltpu.make_async_copy(src_hbm.at[wid, pl.ds((w + 1) * WIN, WIN)],
                                  swin[nslot], wsems[nslot]).wait()
            pltpu.make_async_copy(dst_hbm.at[wid, pl.ds((w + 1) * WIN, WIN)],
                                  dwin[nslot], wsems[nslot]).wait()

    plsc.subcore_barrier()
    pltpu.sync_copy(acc.at[pl.ds(r0, ROWS_PER_TILE)],
                    out_hbm.at[cid, pl.ds(r0, ROWS_PER_TILE)])


_BR = 256
_GRID = N_PAD // _BR


def _degsum_body(dp_ref, o_ref):
    o_ref[...] = jnp.sum(dp_ref[...], axis=0)


def _dinv(d):
    return lax.rsqrt(d + 1.0)


def _u_body(x_ref, w_ref, d_ref, o_ref):
    h = jnp.dot(x_ref[...], w_ref[...], preferred_element_type=jnp.float32)
    o_ref[...] = h * _dinv(d_ref[...])


def _ln_relu(pre, g, beta):
    mu = jnp.mean(pre, axis=1, keepdims=True)
    var = jnp.mean((pre - mu) ** 2, axis=1, keepdims=True)
    return jnp.maximum((pre - mu) * lax.rsqrt(var + 1e-5) * g + beta, 0.0)


def _mid_body(p0_ref, p1_ref, u1_ref, x0_ref, w2_ref, b1_ref, g1_ref,
              be1_ref, d_ref, x1_ref, u2_ref):
    dinv = _dinv(d_ref[...])
    scat = p0_ref[...] + p1_ref[...] - u1_ref[...]
    pre = scat * dinv + b1_ref[...]
    x1 = _ln_relu(pre, g1_ref[...], be1_ref[...]) + x0_ref[...]
    x1_ref[...] = x1
    u2_ref[...] = jnp.dot(x1, w2_ref[...],
                          preferred_element_type=jnp.float32) * dinv


def _final_body(p0_ref, p1_ref, u2_ref, x1_ref, b2_ref, g2_ref, be2_ref,
                d_ref, o_ref):
    dinv = _dinv(d_ref[...])
    scat = p0_ref[...] + p1_ref[...] - u2_ref[...]
    pre = scat * dinv + b2_ref[...]
    o_ref[...] = _ln_relu(pre, g2_ref[...], be2_ref[...]) + x1_ref[...]


def _row_spec():
    return pl.BlockSpec((_BR, D), lambda i: (i, 0))


def _full_spec():
    return pl.BlockSpec((D, D), lambda i: (0, 0))


def _vec_spec():
    return pl.BlockSpec((1, D), lambda i: (0, 0))


def _col_spec():
    return pl.BlockSpec((_BR, 1), lambda i: (i, 0))


_f32 = jnp.float32


def _degsum_call(degp):
    return pl.pallas_call(
        _degsum_body,
        in_specs=[pl.BlockSpec((NW, N_PAD // D, D), lambda: (0, 0, 0))],
        out_specs=pl.BlockSpec((N_PAD // D, D), lambda: (0, 0)),
        out_shape=jax.ShapeDtypeStruct((N_PAD // D, D), _f32),
    )(degp)


def _u_call(xp, W, d):
    return pl.pallas_call(
        _u_body,
        grid=(_GRID,),
        in_specs=[_row_spec(), _full_spec(), _col_spec()],
        out_specs=_row_spec(),
        out_shape=jax.ShapeDtypeStruct((N_PAD, D), _f32),
    )(xp, W, d)


def _mid_call(p0, p1, u1, x0, W2, b1, g1, be1, d):
    return pl.pallas_call(
        _mid_body,
        grid=(_GRID,),
        in_specs=[_row_spec(), _row_spec(), _row_spec(), _row_spec(),
                  _full_spec(), _vec_spec(), _vec_spec(), _vec_spec(),
                  _col_spec()],
        out_specs=[_row_spec(), _row_spec()],
        out_shape=[jax.ShapeDtypeStruct((N_PAD, D), _f32),
                   jax.ShapeDtypeStruct((N_PAD, D), _f32)],
    )(p0, p1, u1, x0, W2, b1, g1, be1, d)


def _final_call(p0, p1, u2, x1, b2, g2, be2, d):
    return pl.pallas_call(
        _final_body,
        grid=(_GRID,),
        in_specs=[_row_spec(), _row_spec(), _row_spec(), _row_spec(),
                  _vec_spec(), _vec_spec(), _vec_spec(), _col_spec()],
        out_specs=_row_spec(),
        out_shape=jax.ShapeDtypeStruct((N_PAD, D), _f32),
    )(p0, p1, u2, x1, b2, g2, be2, d)


def kernel(x, edge_index, W1, b1, g1, beta1, W2, b2, g2, beta2):
    ei = edge_index.astype(jnp.int32)
    pad = E_PAD - N_EDGES
    src = jnp.concatenate([ei[0], jnp.full((pad,), DUMMY, jnp.int32)])
    dst = jnp.concatenate([ei[1], jnp.full((pad,), DUMMY, jnp.int32)])
    src3 = src.reshape(NW, NCHUNK, CHUNK)
    dst3 = dst.reshape(NW, NCHUNK, CHUNK)
    xp = jnp.pad(x, ((0, N_PAD - N_NODES), (0, 0)))

    degp = _deg_kernel(dst3)
    d = _degsum_call(degp.reshape(NW, N_PAD // D, D)).reshape(N_PAD, 1)

    b1r = b1.reshape(1, D)
    g1r = g1.reshape(1, D)
    be1r = beta1.reshape(1, D)
    b2r = b2.reshape(1, D)
    g2r = g2.reshape(1, D)
    be2r = beta2.reshape(1, D)

    u1 = _u_call(xp, W1, d)
    parts1 = _scatter_kernel(u1, src3, dst3)
    x1, u2 = _mid_call(parts1[0], parts1[1], u1, xp, W2, b1r, g1r, be1r, d)
    parts2 = _scatter_kernel(u2, src3, dst3)
    x2 = _final_call(parts2[0], parts2[1], u2, x1, b2r, g2r, be2r, d)
    return x2[:N_NODES]


# speedup vs baseline: 2.6797x; 2.6797x over previous
import functools

import jax
import jax.numpy as jnp
from jax import lax
from jax.experimental import pallas as pl
from jax.experimental.pallas import tpu as pltpu
from jax.experimental.pallas import tpu_sc as plsc

N_NODES = 10000
D = 128
N_EDGES = 320000

NC = 2
NS = 16
NW = NC * NS
COLS = D // NC
CHUNK = 128
NBUF = 2
E_PAD = 327680
NCHUNK_DEG = E_PAD // (NW * CHUNK)
NCHUNK = E_PAD // (NS * CHUNK)
N_PAD = 10240
ROWS_PER_TILE = N_PAD // NS

_mesh = plsc.VectorSubcoreMesh(core_axis_name="c", subcore_axis_name="s")


@functools.partial(
    pl.kernel,
    out_type=jax.ShapeDtypeStruct((NW, N_PAD), jnp.float32),
    mesh=_mesh,
    scratch_types=[
        pltpu.VMEM((NCHUNK_DEG, CHUNK), jnp.int32),
        pltpu.VMEM((N_PAD,), jnp.float32),
    ],
    compiler_params=pltpu.CompilerParams(needs_layout_passes=False),
)
def _deg_kernel(dst_hbm, out_hbm, dst_v, deg_v):
    cid = lax.axis_index("c")
    sid = lax.axis_index("s")
    wid = cid * NS + sid
    pltpu.sync_copy(dst_hbm.at[wid], dst_v)

    zeros16 = jnp.zeros((16,), jnp.float32)

    def _zero(r, carry):
        deg_v[pl.ds(r * 16, 16)] = zeros16
        return carry

    lax.fori_loop(0, N_PAD // 16, _zero, 0)

    ones16 = jnp.ones((16,), jnp.float32)

    def _edges(j, carry):
        def _sub(k, c2):
            idx = dst_v[j, pl.ds(k * 16, 16)]
            plsc.addupdate_scatter(deg_v, [idx], ones16)
            return c2
        return lax.fori_loop(0, CHUNK // 16, _sub, carry)

    lax.fori_loop(0, NCHUNK_DEG, _edges, 0)
    pltpu.sync_copy(deg_v, out_hbm.at[wid])


@functools.partial(
    pl.kernel,
    out_type=jax.ShapeDtypeStruct((NC, N_PAD, COLS), jnp.float32),
    mesh=_mesh,
    scratch_types=[
        pltpu.VMEM((NCHUNK, CHUNK), jnp.int32),
        pltpu.VMEM((NCHUNK, CHUNK), jnp.int32),
        pltpu.VMEM((CHUNK, COLS), jnp.float32),
        pltpu.VMEM((CHUNK, COLS), jnp.float32),
        pltpu.VMEM_SHARED((N_PAD, COLS), jnp.float32),
        pltpu.SemaphoreType.DMA,
        pltpu.SemaphoreType.DMA,
    ],
    compiler_params=pltpu.CompilerParams(needs_layout_passes=False,
                                         use_tc_tiling_on_sc=False),
)
def _scatter_kernel(u_hbm, src_hbm, dst_hbm, out_hbm,
                    src_v, dst_v, r0b, r1b, acc, s0, s1):
    rows = (r0b, r1b)
    sems = (s0, s1)
    cid = lax.axis_index("c")
    sid = lax.axis_index("s")
    pltpu.sync_copy(src_hbm.at[cid, sid], src_v)
    pltpu.sync_copy(dst_hbm.at[sid], dst_v)
    r0 = sid * ROWS_PER_TILE
    pltpu.sync_copy(u_hbm.at[pl.ds(cid * N_PAD + r0, ROWS_PER_TILE)],
                    acc.at[pl.ds(r0, ROWS_PER_TILE)])
    plsc.subcore_barrier()

    for b in range(NBUF):
        pltpu.async_copy(u_hbm.at[src_v.at[b]], rows[b], sems[b])

    def _group(g, carry):
        for b in range(NBUF):
            j = g * NBUF + b
            pltpu.make_async_copy(u_hbm.at[src_v.at[j]], rows[b],
                                  sems[b]).wait()
            pltpu.sync_copy(rows[b], acc.at[dst_v.at[j]], add=True)
            pltpu.async_copy(u_hbm.at[src_v.at[j + NBUF]], rows[b], sems[b])
        return carry

    lax.fori_loop(0, NCHUNK // NBUF - 1, _group, 0)
    for b in range(NBUF):
        j = NCHUNK - NBUF + b
        pltpu.make_async_copy(u_hbm.at[src_v.at[j]], rows[b], sems[b]).wait()
        pltpu.sync_copy(rows[b], acc.at[dst_v.at[j]], add=True)

    plsc.subcore_barrier()
    pltpu.sync_copy(acc.at[pl.ds(r0, ROWS_PER_TILE)],
                    out_hbm.at[cid, pl.ds(r0, ROWS_PER_TILE)])


_BR = 256
_GRID = N_PAD // _BR


def _degsum_body(dp_ref, o_ref):
    o_ref[...] = jnp.sum(dp_ref[...], axis=0)


def _dinv(d):
    return lax.rsqrt(d + 1.0)


def _u_body(x_ref, w_ref, d_ref, lo_ref, hi_ref):
    h = jnp.dot(x_ref[...], w_ref[...], preferred_element_type=jnp.float32)
    u = h * _dinv(d_ref[...])
    lo_ref[...] = u[:, :COLS]
    hi_ref[...] = u[:, COLS:]


def _ln_relu(pre, g, beta):
    mu = jnp.mean(pre, axis=1, keepdims=True)
    var = jnp.mean((pre - mu) ** 2, axis=1, keepdims=True)
    return jnp.maximum((pre - mu) * lax.rsqrt(var + 1e-5) * g + beta, 0.0)


def _mid_body(p0_ref, p1_ref, x0_ref, w2_ref, b1_ref, g1_ref,
              be1_ref, d_ref, x1_ref, lo_ref, hi_ref):
    dinv = _dinv(d_ref[...])
    scat = jnp.concatenate([p0_ref[...], p1_ref[...]], axis=1)
    pre = scat * dinv + b1_ref[...]
    x1 = _ln_relu(pre, g1_ref[...], be1_ref[...]) + x0_ref[...]
    x1_ref[...] = x1
    u2 = jnp.dot(x1, w2_ref[...], preferred_element_type=jnp.float32) * dinv
    lo_ref[...] = u2[:, :COLS]
    hi_ref[...] = u2[:, COLS:]


def _final_body(p0_ref, p1_ref, x1_ref, b2_ref, g2_ref, be2_ref,
                d_ref, o_ref):
    dinv = _dinv(d_ref[...])
    scat = jnp.concatenate([p0_ref[...], p1_ref[...]], axis=1)
    pre = scat * dinv + b2_ref[...]
    o_ref[...] = _ln_relu(pre, g2_ref[...], be2_ref[...]) + x1_ref[...]


def _row_spec():
    return pl.BlockSpec((_BR, D), lambda i: (i, 0))


def _half_spec():
    return pl.BlockSpec((_BR, COLS), lambda i: (i, 0))


def _full_spec():
    return pl.BlockSpec((D, D), lambda i: (0, 0))


def _vec_spec():
    return pl.BlockSpec((1, D), lambda i: (0, 0))


def _col_spec():
    return pl.BlockSpec((_BR, 1), lambda i: (i, 0))


_f32 = jnp.float32


def _degsum_call(degp):
    return pl.pallas_call(
        _degsum_body,
        in_specs=[pl.BlockSpec((NW, N_PAD // D, D), lambda: (0, 0, 0))],
        out_specs=pl.BlockSpec((N_PAD // D, D), lambda: (0, 0)),
        out_shape=jax.ShapeDtypeStruct((N_PAD // D, D), _f32),
    )(degp)


def _u_call(xp, W, d):
    return pl.pallas_call(
        _u_body,
        grid=(_GRID,),
        in_specs=[_row_spec(), _full_spec(), _col_spec()],
        out_specs=[_half_spec(), _half_spec()],
        out_shape=[jax.ShapeDtypeStruct((N_PAD, COLS), _f32),
                   jax.ShapeDtypeStruct((N_PAD, COLS), _f32)],
    )(xp, W, d)


def _mid_call(p0, p1, x0, W2, b1, g1, be1, d):
    return pl.pallas_call(
        _mid_body,
        grid=(_GRID,),
        in_specs=[_half_spec(), _half_spec(), _row_spec(),
                  _full_spec(), _vec_spec(), _vec_spec(), _vec_spec(),
                  _col_spec()],
        out_specs=[_row_spec(), _half_spec(), _half_spec()],
        out_shape=[jax.ShapeDtypeStruct((N_PAD, D), _f32),
                   jax.ShapeDtypeStruct((N_PAD, COLS), _f32),
                   jax.ShapeDtypeStruct((N_PAD, COLS), _f32)],
    )(p0, p1, x0, W2, b1, g1, be1, d)


def _final_call(p0, p1, x1, b2, g2, be2, d):
    return pl.pallas_call(
        _final_body,
        grid=(_GRID,),
        in_specs=[_half_spec(), _half_spec(), _row_spec(),
                  _vec_spec(), _vec_spec(), _vec_spec(), _col_spec()],
        out_specs=_row_spec(),
        out_shape=jax.ShapeDtypeStruct((N_PAD, D), _f32),
    )(p0, p1, x1, b2, g2, be2, d)


def kernel(x, edge_index, W1, b1, g1, beta1, W2, b2, g2, beta2):
    ei = edge_index.astype(jnp.int32)
    pad = E_PAD - N_EDGES
    dummy = N_NODES + jnp.arange(pad, dtype=jnp.int32) % (N_PAD - N_NODES)
    src = jnp.concatenate([ei[0], dummy])
    dst = jnp.concatenate([ei[1], dummy])
    src3 = src.reshape(NS, NCHUNK, CHUNK)
    src_sc = jnp.stack([src3, src3 + N_PAD])
    dst3 = dst.reshape(NS, NCHUNK, CHUNK)
    dst_deg = dst.reshape(NW, NCHUNK_DEG, CHUNK)
    xp = jnp.pad(x, ((0, N_PAD - N_NODES), (0, 0)))

    degp = _deg_kernel(dst_deg)
    d = _degsum_call(degp.reshape(NW, N_PAD // D, D)).reshape(N_PAD, 1)

    b1r = b1.reshape(1, D)
    g1r = g1.reshape(1, D)
    be1r = beta1.reshape(1, D)
    b2r = b2.reshape(1, D)
    g2r = g2.reshape(1, D)
    be2r = beta2.reshape(1, D)

    u1_lo, u1_hi = _u_call(xp, W1, d)
    u1 = jnp.concatenate([u1_lo, u1_hi], axis=0)
    parts1 = _scatter_kernel(u1, src_sc, dst3)
    x1, u2_lo, u2_hi = _mid_call(parts1[0], parts1[1], xp, W2,
                                 b1r, g1r, be1r, d)
    u2 = jnp.concatenate([u2_lo, u2_hi], axis=0)
    parts2 = _scatter_kernel(u2, src_sc, dst3)
    x2 = _final_call(parts2[0], parts2[1], x1, b2r, g2r, be2r, d)
    return x2[:N_NODES]


# speedup vs baseline: 3.1002x; 1.1569x over previous
import functools

import jax
import jax.numpy as jnp
from jax import lax
from jax.experimental import pallas as pl
from jax.experimental.pallas import tpu as pltpu
from jax.experimental.pallas import tpu_sc as plsc

N_NODES = 10000
D = 128
N_EDGES = 320000

NC = 2
NS = 16
NW = NC * NS
COLS = D // NC
CHUNK = 128
NBUF = 4
E_PAD = 327680
NCHUNK_DEG = E_PAD // (NW * CHUNK)
NCHUNK = E_PAD // (NS * CHUNK)
N_PAD = 10240
ROWS_PER_TILE = N_PAD // NS

_mesh = plsc.VectorSubcoreMesh(core_axis_name="c", subcore_axis_name="s")


@functools.partial(
    pl.kernel,
    out_type=jax.ShapeDtypeStruct((NW, N_PAD), jnp.float32),
    mesh=_mesh,
    scratch_types=[
        pltpu.VMEM((NCHUNK_DEG, CHUNK), jnp.int32),
        pltpu.VMEM((N_PAD,), jnp.float32),
    ],
    compiler_params=pltpu.CompilerParams(needs_layout_passes=False),
)
def _deg_kernel(dst_hbm, out_hbm, dst_v, deg_v):
    cid = lax.axis_index("c")
    sid = lax.axis_index("s")
    wid = cid * NS + sid
    pltpu.sync_copy(dst_hbm.at[wid], dst_v)

    zeros16 = jnp.zeros((16,), jnp.float32)

    def _zero(r, carry):
        deg_v[pl.ds(r * 16, 16)] = zeros16
        return carry

    lax.fori_loop(0, N_PAD // 16, _zero, 0)

    ones16 = jnp.ones((16,), jnp.float32)

    def _edges(j, carry):
        def _sub(k, c2):
            idx = dst_v[j, pl.ds(k * 16, 16)]
            plsc.addupdate_scatter(deg_v, [idx], ones16)
            return c2
        return lax.fori_loop(0, CHUNK // 16, _sub, carry)

    lax.fori_loop(0, NCHUNK_DEG, _edges, 0)
    pltpu.sync_copy(deg_v, out_hbm.at[wid])


@functools.partial(
    pl.kernel,
    out_type=jax.ShapeDtypeStruct((NC, N_PAD, COLS), jnp.float32),
    mesh=_mesh,
    scratch_types=[
        pltpu.VMEM((NCHUNK, CHUNK), jnp.int32),
        pltpu.VMEM((NCHUNK, CHUNK), jnp.int32),
        pltpu.VMEM((CHUNK, COLS), jnp.float32),
        pltpu.VMEM((CHUNK, COLS), jnp.float32),
        pltpu.VMEM((CHUNK, COLS), jnp.float32),
        pltpu.VMEM((CHUNK, COLS), jnp.float32),
        pltpu.VMEM_SHARED((N_PAD, COLS), jnp.float32),
        pltpu.SemaphoreType.DMA,
        pltpu.SemaphoreType.DMA,
        pltpu.SemaphoreType.DMA,
        pltpu.SemaphoreType.DMA,
    ],
    compiler_params=pltpu.CompilerParams(needs_layout_passes=False,
                                         use_tc_tiling_on_sc=False),
)
def _scatter_kernel(u_hbm, src_hbm, dst_hbm, out_hbm,
                    src_v, dst_v, r0b, r1b, r2b, r3b, acc, s0, s1, s2, s3):
    rows = (r0b, r1b, r2b, r3b)
    sems = (s0, s1, s2, s3)
    cid = lax.axis_index("c")
    sid = lax.axis_index("s")
    pltpu.sync_copy(src_hbm.at[cid, sid], src_v)
    pltpu.sync_copy(dst_hbm.at[sid], dst_v)
    r0 = sid * ROWS_PER_TILE
    pltpu.sync_copy(u_hbm.at[pl.ds(cid * N_PAD + r0, ROWS_PER_TILE)],
                    acc.at[pl.ds(r0, ROWS_PER_TILE)])
    plsc.subcore_barrier()

    for b in range(NBUF):
        pltpu.async_copy(u_hbm.at[src_v.at[b]], rows[b], sems[b])

    def _group(g, carry):
        for b in range(NBUF):
            j = g * NBUF + b
            pltpu.make_async_copy(u_hbm.at[src_v.at[j]], rows[b],
                                  sems[b]).wait()
            pltpu.sync_copy(rows[b], acc.at[dst_v.at[j]], add=True)
            pltpu.async_copy(u_hbm.at[src_v.at[j + NBUF]], rows[b], sems[b])
        return carry

    lax.fori_loop(0, NCHUNK // NBUF - 1, _group, 0)
    for b in range(NBUF):
        j = NCHUNK - NBUF + b
        pltpu.make_async_copy(u_hbm.at[src_v.at[j]], rows[b], sems[b]).wait()
        pltpu.sync_copy(rows[b], acc.at[dst_v.at[j]], add=True)

    plsc.subcore_barrier()
    pltpu.sync_copy(acc.at[pl.ds(r0, ROWS_PER_TILE)],
                    out_hbm.at[cid, pl.ds(r0, ROWS_PER_TILE)])


_BR = 256
_GRID = N_PAD // _BR


def _degsum_body(dp_ref, o_ref):
    o_ref[...] = jnp.sum(dp_ref[...], axis=0)


def _dinv(d):
    return lax.rsqrt(d + 1.0)


def _u_body(x_ref, w_ref, d_ref, lo_ref, hi_ref):
    h = jnp.dot(x_ref[...], w_ref[...], preferred_element_type=jnp.float32)
    u = h * _dinv(d_ref[...])
    lo_ref[...] = u[:, :COLS]
    hi_ref[...] = u[:, COLS:]


def _ln_relu(pre, g, beta):
    mu = jnp.mean(pre, axis=1, keepdims=True)
    var = jnp.mean((pre - mu) ** 2, axis=1, keepdims=True)
    return jnp.maximum((pre - mu) * lax.rsqrt(var + 1e-5) * g + beta, 0.0)


def _mid_body(p0_ref, p1_ref, x0_ref, w2_ref, b1_ref, g1_ref,
              be1_ref, d_ref, x1_ref, lo_ref, hi_ref):
    dinv = _dinv(d_ref[...])
    scat = jnp.concatenate([p0_ref[...], p1_ref[...]], axis=1)
    pre = scat * dinv + b1_ref[...]
    x1 = _ln_relu(pre, g1_ref[...], be1_ref[...]) + x0_ref[...]
    x1_ref[...] = x1
    u2 = jnp.dot(x1, w2_ref[...], preferred_element_type=jnp.float32) * dinv
    lo_ref[...] = u2[:, :COLS]
    hi_ref[...] = u2[:, COLS:]


def _final_body(p0_ref, p1_ref, x1_ref, b2_ref, g2_ref, be2_ref,
                d_ref, o_ref):
    dinv = _dinv(d_ref[...])
    scat = jnp.concatenate([p0_ref[...], p1_ref[...]], axis=1)
    pre = scat * dinv + b2_ref[...]
    o_ref[...] = _ln_relu(pre, g2_ref[...], be2_ref[...]) + x1_ref[...]


def _row_spec():
    return pl.BlockSpec((_BR, D), lambda i: (i, 0))


def _half_spec():
    return pl.BlockSpec((_BR, COLS), lambda i: (i, 0))


def _full_spec():
    return pl.BlockSpec((D, D), lambda i: (0, 0))


def _vec_spec():
    return pl.BlockSpec((1, D), lambda i: (0, 0))


def _col_spec():
    return pl.BlockSpec((_BR, 1), lambda i: (i, 0))


_f32 = jnp.float32


def _degsum_call(degp):
    return pl.pallas_call(
        _degsum_body,
        in_specs=[pl.BlockSpec((NW, N_PAD // D, D), lambda: (0, 0, 0))],
        out_specs=pl.BlockSpec((N_PAD // D, D), lambda: (0, 0)),
        out_shape=jax.ShapeDtypeStruct((N_PAD // D, D), _f32),
    )(degp)


def _u_call(xp, W, d):
    return pl.pallas_call(
        _u_body,
        grid=(_GRID,),
        in_specs=[_row_spec(), _full_spec(), _col_spec()],
        out_specs=[_half_spec(), _half_spec()],
        out_shape=[jax.ShapeDtypeStruct((N_PAD, COLS), _f32),
                   jax.ShapeDtypeStruct((N_PAD, COLS), _f32)],
    )(xp, W, d)


def _mid_call(p0, p1, x0, W2, b1, g1, be1, d):
    return pl.pallas_call(
        _mid_body,
        grid=(_GRID,),
        in_specs=[_half_spec(), _half_spec(), _row_spec(),
                  _full_spec(), _vec_spec(), _vec_spec(), _vec_spec(),
                  _col_spec()],
        out_specs=[_row_spec(), _half_spec(), _half_spec()],
        out_shape=[jax.ShapeDtypeStruct((N_PAD, D), _f32),
                   jax.ShapeDtypeStruct((N_PAD, COLS), _f32),
                   jax.ShapeDtypeStruct((N_PAD, COLS), _f32)],
    )(p0, p1, x0, W2, b1, g1, be1, d)


def _final_call(p0, p1, x1, b2, g2, be2, d):
    return pl.pallas_call(
        _final_body,
        grid=(_GRID,),
        in_specs=[_half_spec(), _half_spec(), _row_spec(),
                  _vec_spec(), _vec_spec(), _vec_spec(), _col_spec()],
        out_specs=_row_spec(),
        out_shape=jax.ShapeDtypeStruct((N_PAD, D), _f32),
    )(p0, p1, x1, b2, g2, be2, d)


def kernel(x, edge_index, W1, b1, g1, beta1, W2, b2, g2, beta2):
    ei = edge_index.astype(jnp.int32)
    pad = E_PAD - N_EDGES
    dummy = N_NODES + jnp.arange(pad, dtype=jnp.int32) % (N_PAD - N_NODES)
    src = jnp.concatenate([ei[0], dummy])
    dst = jnp.concatenate([ei[1], dummy])
    src3 = src.reshape(NS, NCHUNK, CHUNK)
    src_sc = jnp.stack([src3, src3 + N_PAD])
    dst3 = dst.reshape(NS, NCHUNK, CHUNK)
    dst_deg = dst.reshape(NW, NCHUNK_DEG, CHUNK)
    xp = jnp.pad(x, ((0, N_PAD - N_NODES), (0, 0)))

    degp = _deg_kernel(dst_deg)
    d = _degsum_call(degp.reshape(NW, N_PAD // D, D)).reshape(N_PAD, 1)

    b1r = b1.reshape(1, D)
    g1r = g1.reshape(1, D)
    be1r = beta1.reshape(1, D)
    b2r = b2.reshape(1, D)
    g2r = g2.reshape(1, D)
    be2r = beta2.reshape(1, D)

    u1_lo, u1_hi = _u_call(xp, W1, d)
    u1 = jnp.concatenate([u1_lo, u1_hi], axis=0)
    parts1 = _scatter_kernel(u1, src_sc, dst3)
    x1, u2_lo, u2_hi = _mid_call(parts1[0], parts1[1], xp, W2,
                                 b1r, g1r, be1r, d)
    u2 = jnp.concatenate([u2_lo, u2_hi], axis=0)
    parts2 = _scatter_kernel(u2, src_sc, dst3)
    x2 = _final_call(parts2[0], parts2[1], x1, b2r, g2r, be2r, d)
    return x2[:N_NODES]


# speedup vs baseline: 3.3884x; 1.0930x over previous
import functools

import jax
import jax.numpy as jnp
from jax import lax
from jax.experimental import pallas as pl
from jax.experimental.pallas import tpu as pltpu
from jax.experimental.pallas import tpu_sc as plsc

N_NODES = 10000
D = 128
N_EDGES = 320000

NC = 2
NS = 16
NW = NC * NS
COLS = D // NC
CHUNK = 128
NBUF = 4
E_PAD = 327680
NCHUNK_DEG = E_PAD // (NW * CHUNK)
NCHUNK = E_PAD // (NS * CHUNK)
N_PAD = 10240
ROWS_PER_TILE = N_PAD // NS

_mesh = plsc.VectorSubcoreMesh(core_axis_name="c", subcore_axis_name="s")


@functools.partial(
    pl.kernel,
    out_type=jax.ShapeDtypeStruct((NW, N_PAD), jnp.float32),
    mesh=_mesh,
    scratch_types=[
        pltpu.VMEM((NCHUNK_DEG, CHUNK), jnp.int32),
        pltpu.VMEM((N_PAD,), jnp.float32),
    ],
    compiler_params=pltpu.CompilerParams(needs_layout_passes=False),
)
def _deg_kernel(dst_hbm, out_hbm, dst_v, deg_v):
    cid = lax.axis_index("c")
    sid = lax.axis_index("s")
    wid = cid * NS + sid
    pltpu.sync_copy(dst_hbm.at[wid], dst_v)

    zeros16 = jnp.zeros((16,), jnp.float32)

    def _zero(r, carry):
        deg_v[pl.ds(r * 16, 16)] = zeros16
        return carry

    lax.fori_loop(0, N_PAD // 16, _zero, 0)

    ones16 = jnp.ones((16,), jnp.float32)

    def _edges(j, carry):
        def _sub(k, c2):
            idx = dst_v[j, pl.ds(k * 16, 16)]
            plsc.addupdate_scatter(deg_v, [idx], ones16)
            return c2
        return lax.fori_loop(0, CHUNK // 16, _sub, carry)

    lax.fori_loop(0, NCHUNK_DEG, _edges, 0)
    pltpu.sync_copy(deg_v, out_hbm.at[wid])


@functools.partial(
    pl.kernel,
    out_type=jax.ShapeDtypeStruct((NC, N_PAD, COLS), jnp.float32),
    mesh=_mesh,
    scratch_types=[
        pltpu.VMEM((NCHUNK, CHUNK), jnp.int32),
        pltpu.VMEM((NCHUNK, CHUNK), jnp.int32),
        pltpu.VMEM((CHUNK, COLS), jnp.float32),
        pltpu.VMEM((CHUNK, COLS), jnp.float32),
        pltpu.VMEM((CHUNK, COLS), jnp.float32),
        pltpu.VMEM((CHUNK, COLS), jnp.float32),
        pltpu.VMEM_SHARED((N_PAD, COLS), jnp.float32),
        pltpu.SemaphoreType.DMA,
        pltpu.SemaphoreType.DMA,
        pltpu.SemaphoreType.DMA,
        pltpu.SemaphoreType.DMA,
    ],
    compiler_params=pltpu.CompilerParams(needs_layout_passes=False,
                                         use_tc_tiling_on_sc=False),
)
def _scatter_kernel(u_hbm, src_hbm, dst_hbm, out_hbm,
                    src_v, dst_v, r0b, r1b, r2b, r3b, acc, s0, s1, s2, s3):
    rows = (r0b, r1b, r2b, r3b)
    sems = (s0, s1, s2, s3)
    cid = lax.axis_index("c")
    sid = lax.axis_index("s")
    pltpu.sync_copy(src_hbm.at[cid, sid], src_v)
    pltpu.sync_copy(dst_hbm.at[sid], dst_v)
    r0 = sid * ROWS_PER_TILE
    pltpu.sync_copy(u_hbm.at[pl.ds(cid * N_PAD + r0, ROWS_PER_TILE)],
                    acc.at[pl.ds(r0, ROWS_PER_TILE)])
    plsc.subcore_barrier()

    for b in range(NBUF):
        pltpu.async_copy(u_hbm.at[src_v.at[b]], rows[b], sems[b])

    def _group(g, carry):
        for b in range(NBUF):
            j = g * NBUF + b
            pltpu.make_async_copy(u_hbm.at[src_v.at[j]], rows[b],
                                  sems[b]).wait()
            pltpu.sync_copy(rows[b], acc.at[dst_v.at[j]], add=True)
            pltpu.async_copy(u_hbm.at[src_v.at[j + NBUF]], rows[b], sems[b])
        return carry

    lax.fori_loop(0, NCHUNK // NBUF - 1, _group, 0)
    for b in range(NBUF):
        j = NCHUNK - NBUF + b
        pltpu.make_async_copy(u_hbm.at[src_v.at[j]], rows[b], sems[b]).wait()
        pltpu.sync_copy(rows[b], acc.at[dst_v.at[j]], add=True)

    plsc.subcore_barrier()
    pltpu.sync_copy(acc.at[pl.ds(r0, ROWS_PER_TILE)],
                    out_hbm.at[cid, pl.ds(r0, ROWS_PER_TILE)])


_BR = 256
_GRID = N_PAD // _BR


def _degsum_body(dp_ref, o_ref):
    o_ref[...] = jnp.sum(dp_ref[...], axis=0)


def _dinv(d):
    return lax.rsqrt(d + 1.0)


def _u_body(x_ref, w_ref, d_ref, u_ref):
    h = jnp.dot(x_ref[...], w_ref[...], preferred_element_type=jnp.float32)
    u = h * _dinv(d_ref[...])
    u_ref[0] = u[:, :COLS]
    u_ref[1] = u[:, COLS:]


def _ln_relu(pre, g, beta):
    mu = jnp.mean(pre, axis=1, keepdims=True)
    var = jnp.mean((pre - mu) ** 2, axis=1, keepdims=True)
    return jnp.maximum((pre - mu) * lax.rsqrt(var + 1e-5) * g + beta, 0.0)


def _mid_body(p_ref, x0_ref, w2_ref, b1_ref, g1_ref,
              be1_ref, d_ref, x1_ref, u2_ref):
    dinv = _dinv(d_ref[...])
    scat = jnp.concatenate([p_ref[0], p_ref[1]], axis=1)
    pre = scat * dinv + b1_ref[...]
    x1 = _ln_relu(pre, g1_ref[...], be1_ref[...]) + x0_ref[...]
    x1_ref[...] = x1
    u2 = jnp.dot(x1, w2_ref[...], preferred_element_type=jnp.float32) * dinv
    u2_ref[0] = u2[:, :COLS]
    u2_ref[1] = u2[:, COLS:]


def _final_body(p_ref, x1_ref, b2_ref, g2_ref, be2_ref,
                d_ref, o_ref):
    dinv = _dinv(d_ref[...])
    scat = jnp.concatenate([p_ref[0], p_ref[1]], axis=1)
    pre = scat * dinv + b2_ref[...]
    o_ref[...] = _ln_relu(pre, g2_ref[...], be2_ref[...]) + x1_ref[...]


def _row_spec():
    return pl.BlockSpec((_BR, D), lambda i: (i, 0))


def _stk_spec():
    return pl.BlockSpec((NC, _BR, COLS), lambda i: (0, i, 0))


def _full_spec():
    return pl.BlockSpec((D, D), lambda i: (0, 0))


def _vec_spec():
    return pl.BlockSpec((1, D), lambda i: (0, 0))


def _col_spec():
    return pl.BlockSpec((_BR, 1), lambda i: (i, 0))


_f32 = jnp.float32


def _degsum_call(degp):
    return pl.pallas_call(
        _degsum_body,
        in_specs=[pl.BlockSpec((NW, N_PAD // D, D), lambda: (0, 0, 0))],
        out_specs=pl.BlockSpec((N_PAD // D, D), lambda: (0, 0)),
        out_shape=jax.ShapeDtypeStruct((N_PAD // D, D), _f32),
    )(degp)


def _u_call(xp, W, d):
    return pl.pallas_call(
        _u_body,
        grid=(_GRID,),
        in_specs=[_row_spec(), _full_spec(), _col_spec()],
        out_specs=_stk_spec(),
        out_shape=jax.ShapeDtypeStruct((NC, N_PAD, COLS), _f32),
    )(xp, W, d)


def _mid_call(parts, x0, W2, b1, g1, be1, d):
    return pl.pallas_call(
        _mid_body,
        grid=(_GRID,),
        in_specs=[_stk_spec(), _row_spec(),
                  _full_spec(), _vec_spec(), _vec_spec(), _vec_spec(),
                  _col_spec()],
        out_specs=[_row_spec(), _stk_spec()],
        out_shape=[jax.ShapeDtypeStruct((N_PAD, D), _f32),
                   jax.ShapeDtypeStruct((NC, N_PAD, COLS), _f32)],
    )(parts, x0, W2, b1, g1, be1, d)


def _final_call(parts, x1, b2, g2, be2, d):
    return pl.pallas_call(
        _final_body,
        grid=(_GRID,),
        in_specs=[_stk_spec(), _row_spec(),
                  _vec_spec(), _vec_spec(), _vec_spec(), _col_spec()],
        out_specs=_row_spec(),
        out_shape=jax.ShapeDtypeStruct((N_PAD, D), _f32),
    )(parts, x1, b2, g2, be2, d)


def kernel(x, edge_index, W1, b1, g1, beta1, W2, b2, g2, beta2):
    ei = edge_index.astype(jnp.int32)
    pad = E_PAD - N_EDGES
    dummy = N_NODES + jnp.arange(pad, dtype=jnp.int32) % (N_PAD - N_NODES)
    src = jnp.concatenate([ei[0], dummy])
    dst = jnp.concatenate([ei[1], dummy])
    src3 = src.reshape(NS, NCHUNK, CHUNK)
    src_sc = jnp.stack([src3, src3 + N_PAD])
    dst3 = dst.reshape(NS, NCHUNK, CHUNK)
    dst_deg = dst.reshape(NW, NCHUNK_DEG, CHUNK)
    xp = jnp.pad(x, ((0, N_PAD - N_NODES), (0, 0)))

    degp = _deg_kernel(dst_deg)
    d = _degsum_call(degp.reshape(NW, N_PAD // D, D)).reshape(N_PAD, 1)

    b1r = b1.reshape(1, D)
    g1r = g1.reshape(1, D)
    be1r = beta1.reshape(1, D)
    b2r = b2.reshape(1, D)
    g2r = g2.reshape(1, D)
    be2r = beta2.reshape(1, D)

    u1 = _u_call(xp, W1, d)
    parts1 = _scatter_kernel(u1.reshape(NC * N_PAD, COLS), src_sc, dst3)
    x1, u2 = _mid_call(parts1, xp, W2, b1r, g1r, be1r, d)
    parts2 = _scatter_kernel(u2.reshape(NC * N_PAD, COLS), src_sc, dst3)
    x2 = _final_call(parts2, x1, b2r, g2r, be2r, d)
    return x2[:N_NODES]
